# baseline (speedup 1.0000x reference)
import jax
import jax.numpy as jnp
from jax.experimental import pallas as pl
from jax.experimental.pallas import tpu as pltpu

_B, _N, _C = 8, 2048, 40
_K = 8


def _loss_kernel(tgt_ref, a_ref, b_ref, pred_ref, out_ref):
    pcol = jax.lax.broadcasted_iota(jnp.int32, (1, _C), 1)
    col = jax.lax.broadcasted_iota(jnp.int32, (_N, _K), 1)
    row = jax.lax.broadcasted_iota(jnp.int32, (_K, _N), 0)
    total = jnp.float32(0.0)
    for b in range(_B):
        a = a_ref[b]
        bb = b_ref[b]
        an = jnp.sum(a * a, axis=1, keepdims=True)
        bn = jnp.sum(bb * bb, axis=0, keepdims=True)
        an_hi = an.astype(jnp.bfloat16).astype(jnp.float32)
        an_lo = an - an_hi
        bn_hi = bn.astype(jnp.bfloat16).astype(jnp.float32)
        bn_lo = bn - bn_hi
        lhs = (-2.0 * a
               + jnp.where(col == 3, an_hi, 0.0)
               + jnp.where(col == 4, an_lo, 0.0)
               + jnp.where((col == 5) | (col == 6), 1.0, 0.0))
        rhs = (bb
               + jnp.where((row == 3) | (row == 4), 1.0, 0.0)
               + jnp.where(row == 5, bn_hi, 0.0)
               + jnp.where(row == 6, bn_lo, 0.0))
        g = jax.lax.dot(lhs, rhs, preferred_element_type=jnp.float32)
        m1 = jnp.min(g, axis=1, keepdims=True)
        m2 = jnp.min(g, axis=0, keepdims=True)
        s1 = jnp.sum(jnp.maximum(m1, 0.0))
        s2 = jnp.sum(jnp.maximum(m2, 0.0))
        pv = jnp.sum(jnp.where(pcol == tgt_ref[b], pred_ref[b], 0.0))
        total += (s1 + s2) / (_N * _B) - pv / _B
    out_ref[...] = total.reshape(1, 1)


def kernel(reg, point1, pred, target):
    a3 = jnp.pad(reg, ((0, 0), (0, 0), (0, _K - 3)))
    b3 = jnp.pad(point1, ((0, 0), (0, 0), (0, _K - 3))).transpose(0, 2, 1)
    pred3 = pred.reshape(_B, 1, _C)

    grid_spec = pltpu.PrefetchScalarGridSpec(
        num_scalar_prefetch=1,
        grid=(1,),
        in_specs=[
            pl.BlockSpec((_B, _N, _K), lambda i, tgt: (0, 0, 0)),
            pl.BlockSpec((_B, _K, _N), lambda i, tgt: (0, 0, 0)),
            pl.BlockSpec((_B, 1, _C), lambda i, tgt: (0, 0, 0)),
        ],
        out_specs=pl.BlockSpec((1, 1), lambda i, tgt: (0, 0)),
    )
    out = pl.pallas_call(
        _loss_kernel,
        grid_spec=grid_spec,
        out_shape=jax.ShapeDtypeStruct((1, 1), jnp.float32),
        compiler_params=pltpu.CompilerParams(
            dimension_semantics=("arbitrary",),
        ),
    )(target, a3, b3, pred3)
    return out[0, 0]
